# Initial kernel scaffold; baseline (speedup 1.0000x reference)
import jax
import jax.numpy as jnp
from jax.experimental import pallas as pl


def kernel(x, edge_index, W1, b1, W2, b2, Wl, bl):
    raise NotImplementedError("write your pallas kernel here")


# speedup vs baseline: 32.7690x; 32.7690x over previous
import functools

import jax
import jax.numpy as jnp
from jax import lax
from jax.experimental import pallas as pl
from jax.experimental.pallas import tpu as pltpu
from jax.experimental.pallas import tpu_sc as plsc

N = 10000
E = 320000
D = 128
NC = 2
NS = 16
NW = NC * NS
CH = 128
NCHUNK = 80
EPAD = NW * NCHUNK * CH
NACC = 10240
RPT = NACC // NS
BLK = 1000
GRID = N // BLK

_sc_mesh = plsc.VectorSubcoreMesh(core_axis_name="c", subcore_axis_name="s")


NQ = 4
NCQ = NCHUNK // NQ


@functools.partial(
    pl.kernel,
    out_type=jax.ShapeDtypeStruct((NC, NACC), jnp.float32),
    mesh=_sc_mesh,
    scratch_types=[
        pltpu.VMEM((NCQ, CH), jnp.int32),
        pltpu.VMEM((CH,), jnp.float32),
        pltpu.VMEM((RPT,), jnp.float32),
        pltpu.VMEM_SHARED((NACC,), jnp.float32),
    ],
)
def _sc_degree(cidx_hbm, out_hbm, idx_v, ones_v, zeros_v, acc_sh):
    cid = lax.axis_index("c")
    sid = lax.axis_index("s")
    wid = sid * NC + cid
    for t in range(CH // 16):
        ones_v[pl.ds(16 * t, 16)] = jnp.ones((16,), jnp.float32)
    for t in range(RPT // 16):
        zeros_v[pl.ds(16 * t, 16)] = jnp.zeros((16,), jnp.float32)
    pltpu.sync_copy(zeros_v, acc_sh.at[pl.ds(sid * RPT, RPT)])
    plsc.subcore_barrier()

    for q in range(NQ):
        pltpu.sync_copy(cidx_hbm.at[wid * NQ + q], idx_v)

        def chunk(j, _):
            pltpu.sync_copy(ones_v, acc_sh.at[idx_v.at[j]], add=True)
            return ()

        lax.fori_loop(0, NCQ, chunk, ())
    plsc.subcore_barrier()
    pltpu.sync_copy(acc_sh.at[pl.ds(sid * RPT, RPT)],
                    out_hbm.at[cid, pl.ds(sid * RPT, RPT)])


@functools.partial(
    pl.kernel,
    out_type=jax.ShapeDtypeStruct((NC, NACC, D), jnp.float32),
    mesh=_sc_mesh,
    scratch_types=[
        pltpu.VMEM((2, NCQ, CH), jnp.int32),
        pltpu.VMEM((2, NCQ, CH), jnp.int32),
        pltpu.VMEM((2, CH, D), jnp.float32),
        pltpu.VMEM_SHARED((NACC, D), jnp.float32),
        pltpu.SemaphoreType.DMA,
        pltpu.SemaphoreType.DMA,
        pltpu.SemaphoreType.DMA,
        pltpu.SemaphoreType.DMA,
    ],
)
def _sc_aggregate(g_hbm, ridx_hbm, cidx_hbm, out_hbm,
                  ridx_v, cidx_v, buf_v, acc_sh, sem0, sem1, semi0, semi1):
    cid = lax.axis_index("c")
    sid = lax.axis_index("s")
    wid = sid * NC + cid
    sems = (sem0, sem1)
    isems = (semi0, semi1)

    def stage(q):
        qb = q % 2
        for hbm, vm in ((ridx_hbm, ridx_v), (cidx_hbm, cidx_v)):
            yield pltpu.make_async_copy(hbm.at[wid * NQ + q],
                                        vm.at[qb], isems[qb])

    for cp in stage(0):
        cp.start()
    def zrow(i, _):
        for t in range(D // 16):
            buf_v[0, i, pl.ds(16 * t, 16)] = jnp.zeros((16,), jnp.float32)
        return ()

    lax.fori_loop(0, CH, zrow, ())
    for k in range(RPT // CH):
        pltpu.sync_copy(buf_v.at[0], acc_sh.at[pl.ds(sid * RPT + k * CH, CH)])
    for cp in stage(1):
        cp.start()
    plsc.subcore_barrier()

    for q in range(NQ):
        qb = q % 2
        rq = ridx_v.at[qb]
        cq = cidx_v.at[qb]

        def gather(j, b, rq=rq):
            return pltpu.make_async_copy(g_hbm.at[rq.at[j]], buf_v.at[b],
                                         sems[b])

        def scatter_add(j, b, cq=cq):
            pltpu.sync_copy(buf_v.at[b], acc_sh.at[cq.at[j]], add=True)

        for cp in stage(q):
            cp.wait()
        gather(0, 0).start()
        gather(1, 1).start()

        def step(i, _):
            for b in range(2):
                j = 2 * i + b
                gather(j, b).wait()
                scatter_add(j, b)
                gather(j + 2, b).start()
            return ()

        lax.fori_loop(0, NCQ // 2 - 1, step, ())
        for b in range(2):
            j = NCQ - 2 + b
            gather(j, b).wait()
            scatter_add(j, b)
        if q + 2 < NQ:
            for cp in stage(q + 2):
                cp.start()

    plsc.subcore_barrier()
    pltpu.sync_copy(acc_sh.at[pl.ds(sid * RPT, RPT)],
                    out_hbm.at[cid, pl.ds(sid * RPT, RPT)])


def _tc_first_body(deg_ref, x_ref, w_ref, g_ref, dis_ref):
    deg = deg_ref[0] + deg_ref[1] + 1.0
    dis = lax.rsqrt(deg)
    h = jnp.dot(x_ref[...], w_ref[...], preferred_element_type=jnp.float32)
    g_ref[...] = dis * h
    dis_ref[...] = dis


def _tc_mid_body(sp_ref, g_ref, dis_ref, b_ref, w_ref, g2_ref):
    s = sp_ref[0] + sp_ref[1] + g_ref[...]
    h = jnp.maximum(dis_ref[...] * s + b_ref[...], 0.0)
    g2_ref[...] = dis_ref[...] * jnp.dot(h, w_ref[...],
                                         preferred_element_type=jnp.float32)


def _tc_last_body(sp_ref, g_ref, dis_ref, b_ref, wl_ref, bl_ref, out_ref):
    s = sp_ref[0] + sp_ref[1] + g_ref[...]
    h = jnp.maximum(dis_ref[...] * s + b_ref[...], 0.0)
    out_ref[...] = jnp.dot(h, wl_ref[...],
                           preferred_element_type=jnp.float32) + bl_ref[...]


_tc_first = pl.pallas_call(
    _tc_first_body,
    grid=(GRID,),
    in_specs=[
        pl.BlockSpec((NC, BLK, 1), lambda i: (0, i, 0)),
        pl.BlockSpec((BLK, D), lambda i: (i, 0)),
        pl.BlockSpec((D, D), lambda i: (0, 0)),
    ],
    out_specs=[
        pl.BlockSpec((BLK, D), lambda i: (i, 0)),
        pl.BlockSpec((BLK, 1), lambda i: (i, 0)),
    ],
    out_shape=[
        jax.ShapeDtypeStruct((N, D), jnp.float32),
        jax.ShapeDtypeStruct((N, 1), jnp.float32),
    ],
)

_tc_mid = pl.pallas_call(
    _tc_mid_body,
    grid=(GRID,),
    in_specs=[
        pl.BlockSpec((NC, BLK, D), lambda i: (0, i, 0)),
        pl.BlockSpec((BLK, D), lambda i: (i, 0)),
        pl.BlockSpec((BLK, 1), lambda i: (i, 0)),
        pl.BlockSpec((1, D), lambda i: (0, 0)),
        pl.BlockSpec((D, D), lambda i: (0, 0)),
    ],
    out_specs=pl.BlockSpec((BLK, D), lambda i: (i, 0)),
    out_shape=jax.ShapeDtypeStruct((N, D), jnp.float32),
)

_tc_last = pl.pallas_call(
    _tc_last_body,
    grid=(GRID,),
    in_specs=[
        pl.BlockSpec((NC, BLK, D), lambda i: (0, i, 0)),
        pl.BlockSpec((BLK, D), lambda i: (i, 0)),
        pl.BlockSpec((BLK, 1), lambda i: (i, 0)),
        pl.BlockSpec((1, D), lambda i: (0, 0)),
        pl.BlockSpec((D, 1), lambda i: (0, 0)),
        pl.BlockSpec((1, 1), lambda i: (0, 0)),
    ],
    out_specs=pl.BlockSpec((BLK, 1), lambda i: (i, 0)),
    out_shape=jax.ShapeDtypeStruct((N, 1), jnp.float32),
)


def kernel(x, edge_index, W1, b1, W2, b2, Wl, bl):
    row = edge_index[0].astype(jnp.int32)
    col = edge_index[1].astype(jnp.int32)
    npad = EPAD - E
    ar = jnp.arange(npad, dtype=jnp.int32)
    rpad = (ar * 131) % N
    cpad = N + ar % (NACC - N)
    rr = jnp.concatenate([row, rpad]).reshape(NW * NQ, NCQ, CH)
    cc = jnp.concatenate([col, cpad]).reshape(NW * NQ, NCQ, CH)

    degp = _sc_degree(cc)[:, :, None]
    g1, dis = _tc_first(degp, x, W1.T)
    s1 = _sc_aggregate(g1, rr, cc)
    g2 = _tc_mid(s1, g1, dis, b1[None, :], W2.T)
    s2 = _sc_aggregate(g2, rr, cc)
    return _tc_last(s2, g2, dis, b2[None, :], Wl.T, bl[None, :])
